# baseline (speedup 1.0000x reference)
import functools

import jax
import jax.numpy as jnp
from jax import lax
from jax.experimental import pallas as pl
from jax.experimental.pallas import tpu as pltpu
from jax.experimental.pallas import tpu_sc as plsc

_BATCH = 16384
_STATE_DIM = 32
_CITY_DIM = 64
_NUM_STATES = 1000
_NUM_CITIES = 100000
_CHK = 8192


def _sc_gather_t(state_id, city_id, stab_t, ctab_t):
    info = plsc.get_sparse_core_info()
    mesh = plsc.VectorSubcoreMesh(core_axis_name="c", subcore_axis_name="s")
    n_chunks = _BATCH // _CHK

    @functools.partial(
        pl.kernel,
        mesh=mesh,
        compiler_params=pltpu.CompilerParams(needs_layout_passes=False),
        out_type=[
            jax.ShapeDtypeStruct((_STATE_DIM, _BATCH), jnp.float32),
            jax.ShapeDtypeStruct((_CITY_DIM, _BATCH), jnp.float32),
        ],
        scratch_types=[
            pltpu.VMEM((_NUM_CITIES,), jnp.float32),
            pltpu.VMEM((_NUM_STATES,), jnp.float32),
            pltpu.VMEM((_BATCH,), jnp.int32),
            pltpu.VMEM((_CHK,), jnp.float32),
            pltpu.VMEM_SHARED((2, _BATCH), jnp.int32),
            pltpu.SemaphoreType.DMA,
        ],
    )
    def gather_k(sid_hbm, cid_hbm, stabt_hbm, ctabt_hbm, s_out, c_out,
                 row_v, srow_v, idx_v, out_v, ids_sh, sem):
        sub = lax.axis_index("s")
        wid = sub * info.num_cores + lax.axis_index("c")
        @pl.when(sub == 0)
        def _():
            pltpu.sync_copy(sid_hbm, ids_sh.at[0])
            pltpu.sync_copy(cid_hbm, ids_sh.at[1])
        plsc.subcore_barrier()

        cp0 = pltpu.async_copy(ctabt_hbm.at[2 * wid], row_v, sem)
        pltpu.sync_copy(stabt_hbm.at[wid], srow_v)
        pltpu.sync_copy(ids_sh.at[0], idx_v)

        def gather_chunk(table_ref, k):
            @plsc.parallel_loop(0, _CHK, step=16, unroll=8)
            def _(i):
                iv = idx_v[pl.ds(k * _CHK + i, 16)]
                out_v[pl.ds(i, 16)] = plsc.load_gather(table_ref, [iv])

        def state_chunk(k, carry):
            gather_chunk(srow_v, k)
            pltpu.sync_copy(out_v, s_out.at[wid, pl.ds(k * _CHK, _CHK)])
            return carry

        lax.fori_loop(0, n_chunks, state_chunk, 0)
        pltpu.sync_copy(ids_sh.at[1], idx_v)
        cp0.wait()

        def city_row(r, carry):
            @pl.when(r > 0)
            def _():
                pltpu.sync_copy(ctabt_hbm.at[2 * wid + r], row_v)

            def city_chunk(k, c2):
                gather_chunk(row_v, k)
                pltpu.sync_copy(out_v, c_out.at[2 * wid + r, pl.ds(k * _CHK, _CHK)])
                return c2

            lax.fori_loop(0, n_chunks, city_chunk, 0)
            return carry

        lax.fori_loop(0, 2, city_row, 0)

    return gather_k(state_id, city_id, stab_t, ctab_t)


def _tc_body(s_ref, c_ref, wt_ref, bgb_ref, o_ref):
    ws_t = wt_ref[:, :_STATE_DIM]
    wc_t = wt_ref[:, _STATE_DIM:]
    f = (jnp.dot(ws_t, s_ref[...], preferred_element_type=jnp.float32)
         + jnp.dot(wc_t, c_ref[...], preferred_element_type=jnp.float32)
         + bgb_ref[:, 0:1])
    mean = jnp.mean(f, axis=0, keepdims=True)
    var = jnp.mean((f - mean) * (f - mean), axis=0, keepdims=True)
    o_ref[...] = ((f - mean) * lax.rsqrt(var + 1e-5) * bgb_ref[:, 1:2]
                  + bgb_ref[:, 2:3])


def _tc_fuse(s_emb_t, c_emb_t, W, b, gamma, beta):
    blk = 16384
    bgb = jnp.stack([b, gamma, beta], axis=1)
    w_t = W.T
    out_t = pl.pallas_call(
        _tc_body,
        grid=(_BATCH // blk,),
        in_specs=[
            pl.BlockSpec((_STATE_DIM, blk), lambda i: (0, i)),
            pl.BlockSpec((_CITY_DIM, blk), lambda i: (0, i)),
            pl.BlockSpec((_CITY_DIM, _STATE_DIM + _CITY_DIM), lambda i: (0, 0)),
            pl.BlockSpec((_CITY_DIM, 3), lambda i: (0, 0)),
        ],
        out_specs=pl.BlockSpec((_CITY_DIM, blk), lambda i: (0, i)),
        out_shape=jax.ShapeDtypeStruct((_CITY_DIM, _BATCH), jnp.float32),
    )(s_emb_t, c_emb_t, w_t, bgb)
    return out_t.T


def kernel(state_id, city_id, state_table, city_table, W, b, gamma, beta):
    sid = state_id.astype(jnp.int32)
    cid = city_id.astype(jnp.int32)
    s_emb_t, c_emb_t = _sc_gather_t(sid, cid, state_table.T, city_table.T)
    return _tc_fuse(s_emb_t, c_emb_t, W, b, gamma, beta)


# speedup vs baseline: 1.0412x; 1.0412x over previous
import functools

import jax
import jax.numpy as jnp
from jax import lax
from jax.experimental import pallas as pl
from jax.experimental.pallas import tpu as pltpu
from jax.experimental.pallas import tpu_sc as plsc

_BATCH = 16384
_STATE_DIM = 32
_CITY_DIM = 64
_NUM_STATES = 1000
_NUM_CITIES = 100000
_CHK = 8192


def _sc_gather_t(state_id, city_id, stab_t, ctab_t):
    info = plsc.get_sparse_core_info()
    mesh = plsc.VectorSubcoreMesh(core_axis_name="c", subcore_axis_name="s")
    n_chunks = _BATCH // _CHK

    @functools.partial(
        pl.kernel,
        mesh=mesh,
        compiler_params=pltpu.CompilerParams(needs_layout_passes=False),
        out_type=[
            jax.ShapeDtypeStruct((_STATE_DIM, _BATCH), jnp.float32),
            jax.ShapeDtypeStruct((_CITY_DIM, _BATCH), jnp.float32),
        ],
        scratch_types=[
            pltpu.VMEM((_NUM_CITIES,), jnp.float32),
            pltpu.VMEM((_NUM_STATES,), jnp.float32),
            pltpu.VMEM((_BATCH,), jnp.int32),
            pltpu.VMEM((_CHK,), jnp.float32),
            pltpu.VMEM_SHARED((2, _BATCH), jnp.int32),
            pltpu.SemaphoreType.DMA,
        ],
    )
    def gather_k(sid_hbm, cid_hbm, stabt_hbm, ctabt_hbm, s_out, c_out,
                 row_v, srow_v, idx_v, out_v, ids_sh, sem):
        sub = lax.axis_index("s")
        wid = sub * info.num_cores + lax.axis_index("c")
        cp0 = pltpu.async_copy(ctabt_hbm.at[2 * wid], row_v, sem)
        pltpu.sync_copy(stabt_hbm.at[wid], srow_v)
        @pl.when(sub == 0)
        def _():
            pltpu.sync_copy(sid_hbm, ids_sh.at[0])
            pltpu.sync_copy(cid_hbm, ids_sh.at[1])
        plsc.subcore_barrier()

        pltpu.sync_copy(ids_sh.at[0], idx_v)

        def gather_chunk(table_ref, k):
            @plsc.parallel_loop(0, _CHK, step=16, unroll=8)
            def _(i):
                iv = idx_v[pl.ds(k * _CHK + i, 16)]
                out_v[pl.ds(i, 16)] = plsc.load_gather(table_ref, [iv])

        def state_chunk(k, carry):
            gather_chunk(srow_v, k)
            pltpu.sync_copy(out_v, s_out.at[wid, pl.ds(k * _CHK, _CHK)])
            return carry

        lax.fori_loop(0, n_chunks, state_chunk, 0)
        pltpu.sync_copy(ids_sh.at[1], idx_v)
        cp0.wait()

        def city_row(r, carry):
            @pl.when(r > 0)
            def _():
                pltpu.sync_copy(ctabt_hbm.at[2 * wid + r], row_v)

            def city_chunk(k, c2):
                gather_chunk(row_v, k)
                pltpu.sync_copy(out_v, c_out.at[2 * wid + r, pl.ds(k * _CHK, _CHK)])
                return c2

            lax.fori_loop(0, n_chunks, city_chunk, 0)
            return carry

        lax.fori_loop(0, 2, city_row, 0)

    return gather_k(state_id, city_id, stab_t, ctab_t)


def _tc_body(s_ref, c_ref, wt_ref, bgb_ref, o_ref):
    ws_t = wt_ref[:, :_STATE_DIM]
    wc_t = wt_ref[:, _STATE_DIM:]
    f = (jnp.dot(ws_t, s_ref[...], preferred_element_type=jnp.float32)
         + jnp.dot(wc_t, c_ref[...], preferred_element_type=jnp.float32)
         + bgb_ref[:, 0:1])
    mean = jnp.mean(f, axis=0, keepdims=True)
    var = jnp.mean((f - mean) * (f - mean), axis=0, keepdims=True)
    o_ref[...] = ((f - mean) * lax.rsqrt(var + 1e-5) * bgb_ref[:, 1:2]
                  + bgb_ref[:, 2:3])


def _tc_fuse(s_emb_t, c_emb_t, W, b, gamma, beta):
    blk = 8192
    bgb = jnp.stack([b, gamma, beta], axis=1)
    w_t = W.T
    out_t = pl.pallas_call(
        _tc_body,
        grid=(_BATCH // blk,),
        in_specs=[
            pl.BlockSpec((_STATE_DIM, blk), lambda i: (0, i)),
            pl.BlockSpec((_CITY_DIM, blk), lambda i: (0, i)),
            pl.BlockSpec((_CITY_DIM, _STATE_DIM + _CITY_DIM), lambda i: (0, 0)),
            pl.BlockSpec((_CITY_DIM, 3), lambda i: (0, 0)),
        ],
        out_specs=pl.BlockSpec((_CITY_DIM, blk), lambda i: (0, i)),
        out_shape=jax.ShapeDtypeStruct((_CITY_DIM, _BATCH), jnp.float32),
    )(s_emb_t, c_emb_t, w_t, bgb)
    return out_t.T


def kernel(state_id, city_id, state_table, city_table, W, b, gamma, beta):
    sid = state_id.astype(jnp.int32)
    cid = city_id.astype(jnp.int32)
    s_emb_t, c_emb_t = _sc_gather_t(sid, cid, state_table.T, city_table.T)
    return _tc_fuse(s_emb_t, c_emb_t, W, b, gamma, beta)


# speedup vs baseline: 1.0479x; 1.0064x over previous
import functools

import jax
import jax.numpy as jnp
from jax import lax
from jax.experimental import pallas as pl
from jax.experimental.pallas import tpu as pltpu
from jax.experimental.pallas import tpu_sc as plsc

_BATCH = 16384
_STATE_DIM = 32
_CITY_DIM = 64
_NUM_STATES = 1000
_NUM_CITIES = 100000
_CHK = 8192


def _sc_gather_t(state_id, city_id, stab_t, ctab_t):
    info = plsc.get_sparse_core_info()
    mesh = plsc.VectorSubcoreMesh(core_axis_name="c", subcore_axis_name="s")
    n_chunks = _BATCH // _CHK

    @functools.partial(
        pl.kernel,
        mesh=mesh,
        compiler_params=pltpu.CompilerParams(needs_layout_passes=False),
        out_type=[
            jax.ShapeDtypeStruct((_STATE_DIM, _BATCH), jnp.float32),
            jax.ShapeDtypeStruct((_CITY_DIM, _BATCH), jnp.float32),
        ],
        scratch_types=[
            pltpu.VMEM((_NUM_CITIES,), jnp.float32),
            pltpu.VMEM((_NUM_STATES,), jnp.float32),
            pltpu.VMEM((_BATCH,), jnp.int32),
            pltpu.VMEM((_CHK,), jnp.float32),
            pltpu.VMEM_SHARED((2, _BATCH), jnp.int32),
            pltpu.SemaphoreType.DMA,
        ],
    )
    def gather_k(sid_hbm, cid_hbm, stabt_hbm, ctabt_hbm, s_out, c_out,
                 row_v, srow_v, idx_v, out_v, ids_sh, sem):
        sub = lax.axis_index("s")
        wid = sub * info.num_cores + lax.axis_index("c")
        cp0 = pltpu.async_copy(ctabt_hbm.at[2 * wid], row_v, sem)
        pltpu.sync_copy(stabt_hbm.at[wid], srow_v)
        @pl.when(sub == 0)
        def _():
            pltpu.sync_copy(sid_hbm, ids_sh.at[0])
            pltpu.sync_copy(cid_hbm, ids_sh.at[1])
        plsc.subcore_barrier()

        pltpu.sync_copy(ids_sh.at[0], idx_v)

        def gather_chunk(table_ref, k):
            @plsc.parallel_loop(0, _CHK, step=16, unroll=16)
            def _(i):
                iv = idx_v[pl.ds(k * _CHK + i, 16)]
                out_v[pl.ds(i, 16)] = plsc.load_gather(table_ref, [iv])

        def state_chunk(k, carry):
            gather_chunk(srow_v, k)
            pltpu.sync_copy(out_v, s_out.at[wid, pl.ds(k * _CHK, _CHK)])
            return carry

        lax.fori_loop(0, n_chunks, state_chunk, 0)
        pltpu.sync_copy(ids_sh.at[1], idx_v)
        cp0.wait()

        def city_row(r, carry):
            @pl.when(r > 0)
            def _():
                pltpu.sync_copy(ctabt_hbm.at[2 * wid + r], row_v)

            def city_chunk(k, c2):
                gather_chunk(row_v, k)
                pltpu.sync_copy(out_v, c_out.at[2 * wid + r, pl.ds(k * _CHK, _CHK)])
                return c2

            lax.fori_loop(0, n_chunks, city_chunk, 0)
            return carry

        lax.fori_loop(0, 2, city_row, 0)

    return gather_k(state_id, city_id, stab_t, ctab_t)


def _tc_body(s_ref, c_ref, wt_ref, bgb_ref, o_ref):
    ws_t = wt_ref[:, :_STATE_DIM]
    wc_t = wt_ref[:, _STATE_DIM:]
    f = (jnp.dot(ws_t, s_ref[...], preferred_element_type=jnp.float32)
         + jnp.dot(wc_t, c_ref[...], preferred_element_type=jnp.float32)
         + bgb_ref[:, 0:1])
    mean = jnp.mean(f, axis=0, keepdims=True)
    var = jnp.mean((f - mean) * (f - mean), axis=0, keepdims=True)
    o_ref[...] = ((f - mean) * lax.rsqrt(var + 1e-5) * bgb_ref[:, 1:2]
                  + bgb_ref[:, 2:3])


def _tc_fuse(s_emb_t, c_emb_t, W, b, gamma, beta):
    blk = 8192
    bgb = jnp.stack([b, gamma, beta], axis=1)
    w_t = W.T
    out_t = pl.pallas_call(
        _tc_body,
        grid=(_BATCH // blk,),
        in_specs=[
            pl.BlockSpec((_STATE_DIM, blk), lambda i: (0, i)),
            pl.BlockSpec((_CITY_DIM, blk), lambda i: (0, i)),
            pl.BlockSpec((_CITY_DIM, _STATE_DIM + _CITY_DIM), lambda i: (0, 0)),
            pl.BlockSpec((_CITY_DIM, 3), lambda i: (0, 0)),
        ],
        out_specs=pl.BlockSpec((_CITY_DIM, blk), lambda i: (0, i)),
        out_shape=jax.ShapeDtypeStruct((_CITY_DIM, _BATCH), jnp.float32),
    )(s_emb_t, c_emb_t, w_t, bgb)
    return out_t.T


def kernel(state_id, city_id, state_table, city_table, W, b, gamma, beta):
    sid = state_id.astype(jnp.int32)
    cid = city_id.astype(jnp.int32)
    s_emb_t, c_emb_t = _sc_gather_t(sid, cid, state_table.T, city_table.T)
    return _tc_fuse(s_emb_t, c_emb_t, W, b, gamma, beta)
